# Initial kernel scaffold; baseline (speedup 1.0000x reference)
import jax
import jax.numpy as jnp
from jax.experimental import pallas as pl


def kernel(x, router_mask, gate_w, gate_b, expert_w, expert_b):
    raise NotImplementedError("write your pallas kernel here")


# speedup vs baseline: 2.4261x; 2.4261x over previous
import jax
import jax.numpy as jnp
from jax.experimental import pallas as pl
from jax.experimental.pallas import tpu as pltpu

_B, _S, _D, _E = 2, 2048, 768, 8
_TM = 512


def _moe_body(xf_ref, gw_ref, gb_ref, ew_ref, eb_ref, out_ref, w_scr):
    e = pl.program_id(1)

    @pl.when(e == 0)
    def _compute_routing():
        xb = xf_ref[...]
        logits = jax.lax.dot_general(
            xb.astype(jnp.bfloat16), gw_ref[...].astype(jnp.bfloat16),
            (((1,), (1,)), ((), ())),
            preferred_element_type=jnp.float32,
        ) + gb_ref[...]
        m1 = logits[:, 0:1]
        i1 = jnp.zeros((_TM, 1), jnp.int32)
        for j in range(1, _E):
            c = logits[:, j:j + 1]
            better = c > m1
            i1 = jnp.where(better, j, i1)
            m1 = jnp.maximum(m1, c)
        m2 = jnp.full((_TM, 1), -jnp.inf, jnp.float32)
        i2 = jnp.zeros((_TM, 1), jnp.int32)
        for j in range(_E):
            c = logits[:, j:j + 1]
            cand = jnp.logical_and(i1 != j, c > m2)
            i2 = jnp.where(cand, j, i2)
            m2 = jnp.where(cand, c, m2)
        lane = jax.lax.broadcasted_iota(jnp.int32, (_TM, _E), 1)
        sel = jnp.logical_or(lane == i1, lane == i2)
        w_scr[...] = jnp.where(sel, logits, 0.0)

    lane = jax.lax.broadcasted_iota(jnp.int32, (_TM, _E), 1)
    w_col = jnp.sum(w_scr[...] * (lane == e), axis=1, keepdims=True)
    xb16 = xf_ref[...].astype(jnp.bfloat16)
    wb16 = ew_ref[0].astype(jnp.bfloat16)
    y = jax.lax.dot_general(
        xb16, wb16, (((1,), (1,)), ((), ())),
        preferred_element_type=jnp.float32)
    contrib = w_col * (y + eb_ref[0])

    @pl.when(e == 0)
    def _init():
        out_ref[...] = contrib

    @pl.when(e != 0)
    def _acc():
        out_ref[...] += contrib


def kernel(x, router_mask, gate_w, gate_b, expert_w, expert_b):
    n = _B * _S
    xf = x.reshape(n, _D)
    out = pl.pallas_call(
        _moe_body,
        grid=(n // _TM, _E),
        in_specs=[
            pl.BlockSpec((_TM, _D), lambda t, e: (t, 0)),
            pl.BlockSpec((_E, _D), lambda t, e: (0, 0)),
            pl.BlockSpec((1, _E), lambda t, e: (0, 0)),
            pl.BlockSpec((1, _D, _D), lambda t, e: (e, 0, 0)),
            pl.BlockSpec((1, 1, _D), lambda t, e: (e, 0, 0)),
        ],
        out_specs=pl.BlockSpec((_TM, _D), lambda t, e: (t, 0)),
        out_shape=jax.ShapeDtypeStruct((n, _D), jnp.float32),
        scratch_shapes=[pltpu.VMEM((_TM, _E), jnp.float32)],
        compiler_params=pltpu.CompilerParams(
            dimension_semantics=("parallel", "arbitrary")),
    )(xf, gate_w, gate_b.reshape(1, _E), expert_w,
      expert_b.reshape(_E, 1, _D))
    return out.reshape(_B, _S, _D)


# speedup vs baseline: 3.8977x; 1.6066x over previous
import jax
import jax.numpy as jnp
from jax.experimental import pallas as pl
from jax.experimental.pallas import tpu as pltpu

_B, _S, _D, _E = 2, 2048, 768, 8
_TM = 512


def _moe_body(xb_ref, gw_ref, gb_ref, wflat_ref, eb_ref, out_ref):
    xb = xb_ref[...]
    logits = jax.lax.dot_general(
        xb, gw_ref[...], (((1,), (1,)), ((), ())),
        preferred_element_type=jnp.float32,
    ) + gb_ref[...]

    m1 = logits[:, 0:1]
    i1 = jnp.zeros((_TM, 1), jnp.int32)
    for j in range(1, _E):
        c = logits[:, j:j + 1]
        better = c > m1
        i1 = jnp.where(better, j, i1)
        m1 = jnp.maximum(m1, c)
    m2 = jnp.full((_TM, 1), -jnp.inf, jnp.float32)
    i2 = jnp.zeros((_TM, 1), jnp.int32)
    for j in range(_E):
        c = logits[:, j:j + 1]
        cand = jnp.logical_and(i1 != j, c > m2)
        i2 = jnp.where(cand, j, i2)
        m2 = jnp.where(cand, c, m2)
    lane = jax.lax.broadcasted_iota(jnp.int32, (_TM, _E), 1)
    sel = jnp.logical_or(lane == i1, lane == i2)
    w = jnp.where(sel, logits, 0.0)

    y_all = jax.lax.dot_general(
        xb, wflat_ref[...], (((1,), (1,)), ((), ())),
        preferred_element_type=jnp.float32)

    acc = w[:, 0:1] * (y_all[:, 0:_D] + eb_ref[0:1, :])
    for e in range(1, _E):
        acc += w[:, e:e + 1] * (y_all[:, e * _D:(e + 1) * _D]
                                + eb_ref[e:e + 1, :])
    out_ref[...] = acc


def kernel(x, router_mask, gate_w, gate_b, expert_w, expert_b):
    n = _B * _S
    xb16 = x.reshape(n, _D).astype(jnp.bfloat16)
    gw16 = gate_w.astype(jnp.bfloat16)
    wflat16 = expert_w.reshape(_E * _D, _D).astype(jnp.bfloat16)
    out = pl.pallas_call(
        _moe_body,
        grid=(n // _TM,),
        in_specs=[
            pl.BlockSpec((_TM, _D), lambda t: (t, 0)),
            pl.BlockSpec((_E, _D), lambda t: (0, 0)),
            pl.BlockSpec((1, _E), lambda t: (0, 0)),
            pl.BlockSpec((_E * _D, _D), lambda t: (0, 0)),
            pl.BlockSpec((_E, _D), lambda t: (0, 0)),
        ],
        out_specs=pl.BlockSpec((_TM, _D), lambda t: (t, 0)),
        out_shape=jax.ShapeDtypeStruct((n, _D), jnp.float32),
        compiler_params=pltpu.CompilerParams(
            dimension_semantics=("arbitrary",)),
    )(xb16, gw16, gate_b.reshape(1, _E), wflat16, expert_b)
    return out.reshape(_B, _S, _D)


# speedup vs baseline: 4.8540x; 1.2454x over previous
import jax
import jax.numpy as jnp
from jax.experimental import pallas as pl
from jax.experimental.pallas import tpu as pltpu

_B, _S, _D, _E = 2, 2048, 768, 8
_TM = 512


def _moe_body(x_ref, gw_ref, gb_ref, ew_ref, eb_ref, out_ref, wf_scr):
    t = pl.program_id(0)

    @pl.when(t == 0)
    def _cvt():
        wf_scr[...] = ew_ref[...].astype(jnp.bfloat16)

    xb16 = x_ref[...].astype(jnp.bfloat16)
    logits = jax.lax.dot_general(
        xb16, gw_ref[...].astype(jnp.bfloat16), (((1,), (1,)), ((), ())),
        preferred_element_type=jnp.float32,
    ) + gb_ref[...]

    lane = jax.lax.broadcasted_iota(jnp.int32, (_TM, _E), 1)
    m1 = jnp.max(logits, axis=1, keepdims=True)
    i1 = jnp.min(jnp.where(logits == m1, lane, _E), axis=1, keepdims=True)
    l2 = jnp.where(lane == i1, -jnp.inf, logits)
    m2 = jnp.max(l2, axis=1, keepdims=True)
    i2 = jnp.min(jnp.where(l2 == m2, lane, _E), axis=1, keepdims=True)
    sel = jnp.logical_or(lane == i1, lane == i2)
    w16 = jnp.where(sel, logits, 0.0).astype(jnp.bfloat16)

    acc = jax.lax.dot_general(
        w16, eb_ref[...].astype(jnp.bfloat16), (((1,), (0,)), ((), ())),
        preferred_element_type=jnp.float32)
    for e in range(_E):
        xs = xb16 * w16[:, e:e + 1]
        acc = acc + jax.lax.dot_general(
            xs, wf_scr[e * _D:(e + 1) * _D, :], (((1,), (1,)), ((), ())),
            preferred_element_type=jnp.float32)
    out_ref[...] = acc


def kernel(x, router_mask, gate_w, gate_b, expert_w, expert_b):
    n = _B * _S
    out = pl.pallas_call(
        _moe_body,
        grid=(n // _TM,),
        in_specs=[
            pl.BlockSpec((_TM, _D), lambda t: (t, 0)),
            pl.BlockSpec((_E, _D), lambda t: (0, 0)),
            pl.BlockSpec((1, _E), lambda t: (0, 0)),
            pl.BlockSpec((_E * _D, _D), lambda t: (0, 0)),
            pl.BlockSpec((_E, _D), lambda t: (0, 0)),
        ],
        out_specs=pl.BlockSpec((_TM, _D), lambda t: (t, 0)),
        out_shape=jax.ShapeDtypeStruct((n, _D), jnp.float32),
        scratch_shapes=[pltpu.VMEM((_E * _D, _D), jnp.bfloat16)],
        compiler_params=pltpu.CompilerParams(
            dimension_semantics=("arbitrary",)),
    )(x.reshape(n, _D), gate_w, gate_b.reshape(1, _E),
      expert_w.reshape(_E * _D, _D), expert_b)
    return out.reshape(_B, _S, _D)


# speedup vs baseline: 4.9853x; 1.0270x over previous
import jax
import jax.numpy as jnp
from jax.experimental import pallas as pl
from jax.experimental.pallas import tpu as pltpu

_B, _S, _D, _E = 2, 2048, 768, 8
_TM = 1024


def _moe_body(x_ref, gw_ref, gb_ref, ew_ref, eb_ref, out_ref, wf_scr):
    t = pl.program_id(0)

    @pl.when(t == 0)
    def _cvt():
        wf_scr[...] = ew_ref[...].astype(jnp.bfloat16)

    xb16 = x_ref[...].astype(jnp.bfloat16)
    logits = jax.lax.dot_general(
        xb16, gw_ref[...].astype(jnp.bfloat16), (((1,), (1,)), ((), ())),
        preferred_element_type=jnp.float32,
    ) + gb_ref[...]

    lane = jax.lax.broadcasted_iota(jnp.int32, (_TM, _E), 1)
    m1 = jnp.max(logits, axis=1, keepdims=True)
    i1 = jnp.min(jnp.where(logits == m1, lane, _E), axis=1, keepdims=True)
    l2 = jnp.where(lane == i1, -jnp.inf, logits)
    m2 = jnp.max(l2, axis=1, keepdims=True)
    i2 = jnp.min(jnp.where(l2 == m2, lane, _E), axis=1, keepdims=True)
    sel = jnp.logical_or(lane == i1, lane == i2)
    w16 = jnp.where(sel, logits, 0.0).astype(jnp.bfloat16)

    acc = jax.lax.dot_general(
        w16, eb_ref[...].astype(jnp.bfloat16), (((1,), (0,)), ((), ())),
        preferred_element_type=jnp.float32)
    for e in range(_E):
        xs = xb16 * w16[:, e:e + 1]
        acc = acc + jax.lax.dot_general(
            xs, wf_scr[e * _D:(e + 1) * _D, :], (((1,), (1,)), ((), ())),
            preferred_element_type=jnp.float32)
    out_ref[...] = acc


def kernel(x, router_mask, gate_w, gate_b, expert_w, expert_b):
    n = _B * _S
    out = pl.pallas_call(
        _moe_body,
        grid=(n // _TM,),
        in_specs=[
            pl.BlockSpec((_TM, _D), lambda t: (t, 0)),
            pl.BlockSpec((_E, _D), lambda t: (0, 0)),
            pl.BlockSpec((1, _E), lambda t: (0, 0)),
            pl.BlockSpec((_E * _D, _D), lambda t: (0, 0)),
            pl.BlockSpec((_E, _D), lambda t: (0, 0)),
        ],
        out_specs=pl.BlockSpec((_TM, _D), lambda t: (t, 0)),
        out_shape=jax.ShapeDtypeStruct((n, _D), jnp.float32),
        scratch_shapes=[pltpu.VMEM((_E * _D, _D), jnp.bfloat16)],
        compiler_params=pltpu.CompilerParams(
            dimension_semantics=("arbitrary",)),
    )(x.reshape(n, _D), gate_w, gate_b.reshape(1, _E),
      expert_w.reshape(_E * _D, _D), expert_b)
    return out.reshape(_B, _S, _D)
